# baseline (speedup 1.0000x reference)
import jax
import jax.numpy as jnp
from jax.experimental import pallas as pl
from jax.experimental.pallas import tpu as pltpu


def _dot_nt(a, b):
    return jax.lax.dot_general(
        a, b, (((1,), (1,)), ((), ())), preferred_element_type=jnp.float32
    )


def _gnn_fwd(
    data_ref, graph_hbm,
    fw0, fb0, f1w0, f1b0,
    fw1, fb1, f1w1, f1b1,
    fw2, fb2, f1w2, f1b2,
    gw, gb, cw0, cb0, cw1, cb1,
    out_ref,
    graph_vmem, sg,
):
    cpg = pltpu.make_async_copy(graph_hbm, graph_vmem, sg)
    cpg.start()

    hs = []
    h = jnp.maximum(_dot_nt(data_ref[0], fw0[...]) + fb0[...], 0.0)
    hs.append(jnp.maximum(_dot_nt(h, f1w0[...]) + f1b0[...], 0.0))
    h = jnp.maximum(_dot_nt(data_ref[1], fw1[...]) + fb1[...], 0.0)
    hs.append(jnp.maximum(_dot_nt(h, f1w1[...]) + f1b1[...], 0.0))
    h = jnp.maximum(_dot_nt(data_ref[2], fw2[...]) + fb2[...], 0.0)
    hs.append(jnp.maximum(_dot_nt(h, f1w2[...]) + f1b2[...], 0.0))

    mm = jnp.concatenate(hs, axis=1)
    xw = _dot_nt(mm, gw[...])

    cpg.wait()
    graph = graph_vmem[...]
    deg = jnp.sum(graph, axis=0) + 1.0
    dis = jnp.where(deg > 0, jax.lax.rsqrt(jnp.maximum(deg, 1e-12)), 0.0)
    sx = xw * dis[:, None]
    y = jax.lax.dot_general(
        graph, sx, (((0,), (0,)), ((), ())), preferred_element_type=jnp.float32
    )
    g = dis[:, None] * (y + sx) + gb[...]

    z = jnp.concatenate([mm, g], axis=1)
    h = _dot_nt(z, cw0[...]) + cb0[...]
    h = jnp.where(h >= 0, h, 0.01 * h)
    out_ref[...] = _dot_nt(h, cw1[...]) + cb1[...]


def kernel(data_list, graph, fc_w0, fc_b0, fc1_w0, fc1_b0, fc_w1, fc_b1,
           fc1_w1, fc1_b1, fc_w2, fc_b2, fc1_w2, fc1_b2, gcn_w, gcn_b,
           cls_w0, cls_b0, cls_w1, cls_b1):
    V, N, D = data_list.shape
    H0 = gcn_b.shape[0]
    C = cls_w1.shape[0]
    vmem = pl.BlockSpec(memory_space=pltpu.VMEM)
    return pl.pallas_call(
        _gnn_fwd,
        grid=(1,),
        in_specs=[
            pl.BlockSpec((V, N, D), lambda i: (0, 0, 0)),
            pl.BlockSpec(memory_space=pl.ANY),
        ] + [vmem] * 18,
        out_specs=pl.BlockSpec((N, C), lambda i: (0, 0)),
        out_shape=jax.ShapeDtypeStruct((N, C), jnp.float32),
        scratch_shapes=[
            pltpu.VMEM((N, N), jnp.float32),
            pltpu.SemaphoreType.DMA,
        ],
    )(data_list, graph, fc_w0, fc_b0, fc1_w0, fc1_b0, fc_w1,
      fc_b1, fc1_w1, fc1_b1, fc_w2, fc_b2, fc1_w2, fc1_b2, gcn_w, gcn_b,
      cls_w0, cls_b0, cls_w1, cls_b1)


# speedup vs baseline: 1.0152x; 1.0152x over previous
import jax
import jax.numpy as jnp
from jax.experimental import pallas as pl
from jax.experimental.pallas import tpu as pltpu


def _dot_nt(a, b):
    return jax.lax.dot_general(
        a, b, (((1,), (1,)), ((), ())), preferred_element_type=jnp.float32
    )


def _gnn_fwd(
    data01_ref, data_hbm, graph_hbm,
    fw0, fb0, f1w0, f1b0,
    fw1, fb1, f1w1, f1b1,
    fw2, fb2, f1w2, f1b2,
    gw, gb, cw0, cb0, cw1, cb1,
    out_ref,
    d2, graph_vmem, s2, sg,
):
    cp2 = pltpu.make_async_copy(data_hbm.at[2], d2, s2)
    cpg = pltpu.make_async_copy(graph_hbm, graph_vmem, sg)
    cp2.start()
    cpg.start()

    hs = []
    h = jnp.maximum(_dot_nt(data01_ref[0], fw0[...]) + fb0[...], 0.0)
    hs.append(jnp.maximum(_dot_nt(h, f1w0[...]) + f1b0[...], 0.0))
    h = jnp.maximum(_dot_nt(data01_ref[1], fw1[...]) + fb1[...], 0.0)
    hs.append(jnp.maximum(_dot_nt(h, f1w1[...]) + f1b1[...], 0.0))
    cp2.wait()
    h = jnp.maximum(_dot_nt(d2[...], fw2[...]) + fb2[...], 0.0)
    hs.append(jnp.maximum(_dot_nt(h, f1w2[...]) + f1b2[...], 0.0))

    mm = jnp.concatenate(hs, axis=1)
    xw = _dot_nt(mm, gw[...])

    cpg.wait()
    graph = graph_vmem[...]
    deg = jnp.sum(graph, axis=0) + 1.0
    dis = jnp.where(deg > 0, jax.lax.rsqrt(jnp.maximum(deg, 1e-12)), 0.0)
    sx = xw * dis[:, None]
    y = jax.lax.dot_general(
        graph, sx, (((0,), (0,)), ((), ())), preferred_element_type=jnp.float32
    )
    g = dis[:, None] * (y + sx) + gb[...]

    z = jnp.concatenate([mm, g], axis=1)
    h = _dot_nt(z, cw0[...]) + cb0[...]
    h = jnp.where(h >= 0, h, 0.01 * h)
    out_ref[...] = _dot_nt(h, cw1[...]) + cb1[...]


def kernel(data_list, graph, fc_w0, fc_b0, fc1_w0, fc1_b0, fc_w1, fc_b1,
           fc1_w1, fc1_b1, fc_w2, fc_b2, fc1_w2, fc1_b2, gcn_w, gcn_b,
           cls_w0, cls_b0, cls_w1, cls_b1):
    V, N, D = data_list.shape
    H0 = gcn_b.shape[0]
    C = cls_w1.shape[0]
    vmem = pl.BlockSpec(memory_space=pltpu.VMEM)
    return pl.pallas_call(
        _gnn_fwd,
        grid=(1,),
        in_specs=[
            pl.BlockSpec((2, N, D), lambda i: (0, 0, 0)),
            pl.BlockSpec(memory_space=pl.ANY),
            pl.BlockSpec(memory_space=pl.ANY),
        ] + [vmem] * 18,
        out_specs=pl.BlockSpec((N, C), lambda i: (0, 0)),
        out_shape=jax.ShapeDtypeStruct((N, C), jnp.float32),
        scratch_shapes=[
            pltpu.VMEM((N, D), jnp.float32),
            pltpu.VMEM((N, N), jnp.float32),
            pltpu.SemaphoreType.DMA,
            pltpu.SemaphoreType.DMA,
        ],
    )(data_list, data_list, graph, fc_w0, fc_b0, fc1_w0, fc1_b0, fc_w1,
      fc_b1, fc1_w1, fc1_b1, fc_w2, fc_b2, fc1_w2, fc1_b2, gcn_w, gcn_b,
      cls_w0, cls_b0, cls_w1, cls_b1)
